# Initial kernel scaffold; baseline (speedup 1.0000x reference)
import jax
import jax.numpy as jnp
from jax.experimental import pallas as pl


def kernel(xyz1, xyz2, points1, points2, W1, b1, g1, beta1, W2, b2, g2, beta2):
    raise NotImplementedError("write your pallas kernel here")


# speedup vs baseline: 15.7056x; 15.7056x over previous
import jax
import jax.numpy as jnp
from jax.experimental import pallas as pl
from jax.experimental.pallas import tpu as pltpu

_B, _N, _M = 8, 4096, 1024
_C1, _C2 = 128, 256
_H1, _H2 = 256, 128
_BN = 512
_NB = _N // _BN

_interpret = False


def _dot(a, b):
    return jax.lax.dot_general(
        a, b, (((1,), (0,)), ((), ())),
        precision=jax.lax.Precision.HIGHEST,
        preferred_element_type=jnp.float32)


def _k1_body(xyz1_ref, xyz2t_ref, p1_ref, p2_ref, w1at_ref, w1bt_ref, b1_ref,
             h1_ref, stats_ref, p2w_ref):
    b = pl.program_id(0)
    nb = pl.program_id(1)

    @pl.when(nb == 0)
    def _():
        p2w_ref[...] = _dot(p2_ref[0], w1bt_ref[...])

    x1 = xyz1_ref[0]
    x2t = xyz2t_ref[0]

    mm = jax.lax.dot_general(x1, x2t, (((1,), (0,)), ((), ())),
                             preferred_element_type=jnp.float32)
    s1 = x1[:, 0:1] * x1[:, 0:1] + x1[:, 1:2] * x1[:, 1:2] + x1[:, 2:3] * x1[:, 2:3]
    s2 = x2t[0:1, :] * x2t[0:1, :] + x2t[1:2, :] * x2t[1:2, :] + x2t[2:3, :] * x2t[2:3, :]
    d = -2.0 * mm
    d = d + s1
    d = d + s2

    iota = jax.lax.broadcasted_iota(jnp.int32, (_BN, _M), 1)
    big = jnp.float32(jnp.inf)
    recips = []
    masks = []
    for _ in range(3):
        m = jnp.min(d, axis=1, keepdims=True)
        col = jnp.min(jnp.where(d == m, iota, _M), axis=1, keepdims=True)
        mask = iota == col
        d = jnp.where(mask, big, d)
        recips.append(1.0 / (m + 1e-8))
        masks.append(mask)
    norm = recips[0] + recips[1] + recips[2]
    s = jnp.zeros((_BN, _M), jnp.float32)
    for k in range(3):
        s = s + jnp.where(masks[k], recips[k] / norm, 0.0)

    h1 = _dot(s, p2w_ref[...]) + _dot(p1_ref[0], w1at_ref[...]) + b1_ref[...]
    h1_ref[0] = h1

    @pl.when((b == 0) & (nb == 0))
    def _():
        stats_ref[...] = jnp.zeros((8, _H1), jnp.float32)

    stats_ref[0:1, :] += jnp.sum(h1, axis=0, keepdims=True)
    stats_ref[1:2, :] += jnp.sum(h1 * h1, axis=0, keepdims=True)


def _bn_scale_shift(stats, g, beta):
    cnt = jnp.float32(_B * _N)
    mean = stats[0:1, :] / cnt
    var = stats[1:2, :] / cnt - mean * mean
    rstd = jax.lax.rsqrt(var + 1e-5)
    scale = rstd * g
    shift = beta - mean * scale
    return scale, shift


def _k2_body(h1_ref, stats1_ref, g1_ref, beta1_ref, w2t_ref, b2_ref,
             h2_ref, stats_ref):
    b = pl.program_id(0)
    nb = pl.program_id(1)
    scale, shift = _bn_scale_shift(stats1_ref[...], g1_ref[...], beta1_ref[...])
    hn = jnp.maximum(h1_ref[0] * scale + shift, 0.0)
    h2 = _dot(hn, w2t_ref[...]) + b2_ref[...]
    h2_ref[0] = h2

    @pl.when((b == 0) & (nb == 0))
    def _():
        stats_ref[...] = jnp.zeros((8, _H2), jnp.float32)

    stats_ref[0:1, :] += jnp.sum(h2, axis=0, keepdims=True)
    stats_ref[1:2, :] += jnp.sum(h2 * h2, axis=0, keepdims=True)


def _k3_body(h2_ref, stats2_ref, g2_ref, beta2_ref, out_ref):
    scale, shift = _bn_scale_shift(stats2_ref[...], g2_ref[...], beta2_ref[...])
    out_ref[0] = jnp.maximum(h2_ref[0] * scale + shift, 0.0)


def kernel(xyz1, xyz2, points1, points2, W1, b1, g1, beta1, W2, b2, g2, beta2):
    xyz2t = jnp.transpose(xyz2, (0, 2, 1))
    w1at = jnp.transpose(W1[:, :_C1])
    w1bt = jnp.transpose(W1[:, _C1:])
    w2t = jnp.transpose(W2)
    b1r = b1.reshape(1, _H1)
    b2r = b2.reshape(1, _H2)
    g1r = g1.reshape(1, _H1)
    beta1r = beta1.reshape(1, _H1)
    g2r = g2.reshape(1, _H2)
    beta2r = beta2.reshape(1, _H2)

    grid = (_B, _NB)
    h1, stats1 = pl.pallas_call(
        _k1_body,
        grid=grid,
        in_specs=[
            pl.BlockSpec((1, _BN, 3), lambda b, n: (b, n, 0)),
            pl.BlockSpec((1, 3, _M), lambda b, n: (b, 0, 0)),
            pl.BlockSpec((1, _BN, _C1), lambda b, n: (b, n, 0)),
            pl.BlockSpec((1, _M, _C2), lambda b, n: (b, 0, 0)),
            pl.BlockSpec((_C1, _H1), lambda b, n: (0, 0)),
            pl.BlockSpec((_C2, _H1), lambda b, n: (0, 0)),
            pl.BlockSpec((1, _H1), lambda b, n: (0, 0)),
        ],
        out_specs=[
            pl.BlockSpec((1, _BN, _H1), lambda b, n: (b, n, 0)),
            pl.BlockSpec((8, _H1), lambda b, n: (0, 0)),
        ],
        out_shape=[
            jax.ShapeDtypeStruct((_B, _N, _H1), jnp.float32),
            jax.ShapeDtypeStruct((8, _H1), jnp.float32),
        ],
        scratch_shapes=[pltpu.VMEM((_M, _H1), jnp.float32)],
        interpret=_interpret,
    )(xyz1, xyz2t, points1, points2, w1at, w1bt, b1r)

    h2, stats2 = pl.pallas_call(
        _k2_body,
        grid=grid,
        in_specs=[
            pl.BlockSpec((1, _BN, _H1), lambda b, n: (b, n, 0)),
            pl.BlockSpec((8, _H1), lambda b, n: (0, 0)),
            pl.BlockSpec((1, _H1), lambda b, n: (0, 0)),
            pl.BlockSpec((1, _H1), lambda b, n: (0, 0)),
            pl.BlockSpec((_H1, _H2), lambda b, n: (0, 0)),
            pl.BlockSpec((1, _H2), lambda b, n: (0, 0)),
        ],
        out_specs=[
            pl.BlockSpec((1, _BN, _H2), lambda b, n: (b, n, 0)),
            pl.BlockSpec((8, _H2), lambda b, n: (0, 0)),
        ],
        out_shape=[
            jax.ShapeDtypeStruct((_B, _N, _H2), jnp.float32),
            jax.ShapeDtypeStruct((8, _H2), jnp.float32),
        ],
        interpret=_interpret,
    )(h1, stats1, g1r, beta1r, w2t, b2r)

    out = pl.pallas_call(
        _k3_body,
        grid=grid,
        in_specs=[
            pl.BlockSpec((1, _BN, _H2), lambda b, n: (b, n, 0)),
            pl.BlockSpec((8, _H2), lambda b, n: (0, 0)),
            pl.BlockSpec((1, _H2), lambda b, n: (0, 0)),
            pl.BlockSpec((1, _H2), lambda b, n: (0, 0)),
        ],
        out_specs=pl.BlockSpec((1, _BN, _H2), lambda b, n: (b, n, 0)),
        out_shape=jax.ShapeDtypeStruct((_B, _N, _H2), jnp.float32),
        interpret=_interpret,
    )(h2, stats2, g2r, beta2r)

    return out
